# baseline (speedup 1.0000x reference)
import jax
import jax.numpy as jnp
from jax import lax
from jax.experimental import pallas as pl
from jax.experimental.pallas import tpu as pltpu
from jax.experimental.pallas import tpu_sc as plsc

_B = 16384
_C = 5
_D = 64
_V = 1000000
_NW = 32
_BW = _B // _NW
_CHV = 512
_NCHUNK = _V // _CHV + 1
_TSLOT = (_NCHUNK + _NW - 1) // _NW
_BROW = 256
_BCAP = 240
_SLOTS = 64
_EB = 64

_PARAMS = dict(
    compiler_params=pltpu.CompilerParams(use_tc_tiling_on_sc=True,
                                         needs_layout_passes=False),
)


def _make_sweep(n_idx):
    nseg = n_idx // 4096
    dump = n_idx

    def body(tT, tailT, idx_hbm, gat_hbm,
             segbuf, bv, bp, counts, sv, sp, chunk, chunk_b, staging,
             posrow, sem, sem_a, sem_b):
        wid = lax.axis_index("s") * 2 + lax.axis_index("c")
        iota = lax.iota(jnp.int32, 16)
        czero = iota * 0
        lane0 = iota == 0

        for q in range(5):
            counts[pl.ds(16 * q, 16)] = czero

        def scan_seg(seg):
            pltpu.sync_copy(idx_hbm.at[pl.ds(seg * 4096, 4096)], segbuf)

            def scan_vec(i, carry):
                v16 = segbuf[pl.ds(16 * i, 16)]
                cid16 = lax.shift_right_logical(v16, 9)
                mine = lax.bitwise_and(cid16, 31) == wid

                @pl.when(jnp.any(mine))
                def _():
                    pos16 = iota + (seg * 4096 + 16 * i)
                    cnt = jnp.squeeze(lax.slice(
                        plsc.all_reduce_population_count(mine), (0,), (1,)))
                    plsc.store_compressed(sv.at[pl.ds(0, 16)], v16,
                                          mask=mine)
                    plsc.store_compressed(sp.at[pl.ds(0, 16)], pos16,
                                          mask=mine)

                    def app(j, carry2):
                        vv = jnp.squeeze(lax.slice(
                            sv[pl.ds(j, 16)], (0,), (1,)))
                        pp = jnp.squeeze(lax.slice(
                            sp[pl.ds(j, 16)], (0,), (1,)))
                        t = jnp.minimum(
                            lax.shift_right_logical(vv, 14), _TSLOT - 1)
                        off = jnp.minimum(jnp.squeeze(lax.slice(
                            counts[pl.ds(t, 16)], (0,), (1,))), _BCAP)
                        okv = lane0 & (off < _BCAP)
                        slotpos = czero + (t * _BROW + off)
                        plsc.store_scatter(
                            bv, [slotpos],
                            czero + lax.bitwise_and(vv, _CHV - 1), mask=okv)
                        plsc.store_scatter(bp, [slotpos], czero + pp,
                                           mask=okv)
                        plsc.store_scatter(counts, [czero + t],
                                           czero + (off + 1), mask=okv)
                        return carry2

                    lax.fori_loop(0, cnt, app, 0)

                return carry

            lax.fori_loop(0, 256, scan_vec, 0)

        for seg in range(nseg):
            scan_seg(seg)

        dump0 = dump + wid * _SLOTS

        def reset_posrow():
            for q in range(_SLOTS // 16):
                posrow[0, pl.ds(16 * q, 16)] = iota + (dump0 + 16 * q)

        reset_posrow()

        def process(t, buf, slot0, enable):
            tcnt = jnp.squeeze(lax.slice(
                counts[pl.ds(t, 16)], (0,), (1,)))
            tcnt = lax.select(enable, tcnt, 0)

            def lane(j, slot):
                vcol = czero + lax.bitwise_and(jnp.squeeze(lax.slice(
                    bv[pl.ds(t * _BROW + j, 16)], (0,), (1,))), _CHV - 1)
                pos = jnp.clip(jnp.squeeze(lax.slice(
                    bp[pl.ds(t * _BROW + j, 16)], (0,), (1,))),
                    0, n_idx - 1)
                for q in range(4):
                    cv = plsc.load_gather(buf, [iota + 16 * q, vcol])
                    staging[slot, pl.ds(16 * q, 16)] = cv
                plsc.store_scatter(posrow.at[0], [czero + slot],
                                   czero + pos, mask=lane0)
                slot = slot + 1

                @pl.when(slot == _SLOTS)
                def _():
                    pltpu.async_copy(staging, gat_hbm.at[posrow.at[0]],
                                     sem).wait()
                    reset_posrow()

                return lax.select(slot == _SLOTS, 0, slot)

            return lax.fori_loop(0, tcnt, lane, slot0)

        def start_main(t, buf, dsem):
            cid = wid + _NW * t

            @pl.when(cid < _NCHUNK - 1)
            def _():
                off = pl.multiple_of(cid * _CHV, 128)
                pltpu.async_copy(tT.at[:, pl.ds(off, _CHV)], buf, dsem)

        def wait_main(t, buf, dsem):
            cid = wid + _NW * t

            @pl.when(cid < _NCHUNK - 1)
            def _():
                pltpu.make_async_copy(
                    tT.at[:, pl.ds(0, _CHV)], buf, dsem).wait()

        start_main(0, chunk, sem_a)

        def do_pair(p, slot):
            t0 = 2 * p
            start_main(t0 + 1, chunk_b, sem_b)
            wait_main(t0, chunk, sem_a)
            slot = process(t0, chunk, slot,
                           wid + _NW * t0 < _NCHUNK - 1)
            start_main(t0 + 2, chunk, sem_a)
            wait_main(t0 + 1, chunk_b, sem_b)
            slot = process(t0 + 1, chunk_b, slot,
                           wid + _NW * (t0 + 1) < _NCHUNK - 1)
            return slot

        slot = lax.fori_loop(0, _TSLOT // 2, do_pair, 0)

        tail_cid = _NCHUNK - 1
        is_tail_owner = wid == tail_cid % _NW

        @pl.when(is_tail_owner)
        def _():
            pltpu.sync_copy(tailT, chunk.at[:, pl.ds(0, 128)])

        slot = process(_TSLOT - 1, chunk, slot, is_tail_owner)

        pltpu.async_copy(staging, gat_hbm.at[posrow.at[0]], sem).wait()

    return body


def _kb_body(gctx_hbm, gtgt_hbm, wb_hbm, out_hbm,
             ctx_rows, tgt_rows, out_v, wb_v, sem):
    wid = lax.axis_index("s") * 2 + lax.axis_index("c")
    base = wid * _BW
    iota = lax.iota(jnp.int32, 16)

    pltpu.sync_copy(wb_hbm, wb_v)
    tidx_pats = []
    wpats = []
    for v in range(20):
        k = iota + 16 * v
        tidx_pats.append(k // 5)
        wpats.append(plsc.load_gather(wb_v, [k % 5]))
    bvec = plsc.load_gather(wb_v, [iota * 0 + _C])

    for ch in range(_BW // _EB):
        cb = base + ch * _EB
        cp1 = pltpu.async_copy(gctx_hbm.at[pl.ds(cb * _C, _EB * _C)],
                               ctx_rows, sem)
        cp2 = pltpu.async_copy(gtgt_hbm.at[pl.ds(cb, _EB)], tgt_rows, sem)
        cp1.wait()
        cp2.wait()

        def body(e, carry):
            full_e = iota * 0 + e
            acc = jnp.zeros((16,), jnp.float32)
            for v in range(20):
                c_o, q = divmod(v, 4)
                cvec = ctx_rows[_C * e + c_o, pl.ds(16 * q, 16)]
                texp = plsc.load_gather(tgt_rows, [full_e, tidx_pats[v]])
                acc = acc + cvec * wpats[v] * texp
            s = jnp.zeros((16,), jnp.float32) + jnp.sum(acc)
            plsc.store_scatter(out_v, [full_e + ch * _EB], s,
                               mask=iota == 0)
            return carry

        lax.fori_loop(0, _EB, body, 0)

    for i in range(_BW // 16):
        x = out_v[pl.ds(16 * i, 16)] + bvec
        out_v[pl.ds(16 * i, 16)] = 1.0 / (1.0 + jnp.exp(-x))
    pltpu.sync_copy(out_v, out_hbm.at[pl.ds(base, _BW)])


@jax.jit
def kernel(context_input, target_input, context_table, target_table,
           W_dense, b_dense):
    ctx_idx = context_input.reshape(_B * _C)
    tgt_idx = target_input.reshape(_B)
    wb = jnp.concatenate([W_dense.reshape(_C), b_dense,
                          jnp.zeros((2,), jnp.float32)])
    ctxT = context_table.T
    tgtT = target_table.T
    tail_c = jnp.pad(ctxT[:, _V - _V % _CHV:], ((0, 0), (0, 64)))
    tail_t = jnp.pad(tgtT[:, _V - _V % _CHV:], ((0, 0), (0, 64)))

    mesh = plsc.VectorSubcoreMesh(core_axis_name="c", subcore_axis_name="s",
                                  num_cores=2, num_subcores=16)

    def sweep(n_idx):
        return pl.kernel(
            _make_sweep(n_idx),
            out_type=jax.ShapeDtypeStruct((n_idx + _NW * _SLOTS, 128),
                                          jnp.float32),
            mesh=mesh,
            scratch_types=[
                pltpu.VMEM((4096,), jnp.int32),
                pltpu.VMEM((_TSLOT * _BROW,), jnp.int32),
                pltpu.VMEM((_TSLOT * _BROW,), jnp.int32),
                pltpu.VMEM((80,), jnp.int32),
                pltpu.VMEM((32,), jnp.int32),
                pltpu.VMEM((32,), jnp.int32),
                pltpu.VMEM((_D, _CHV), jnp.float32),
                pltpu.VMEM((_D, _CHV), jnp.float32),
                pltpu.VMEM((_SLOTS, 128), jnp.float32),
                pltpu.VMEM((1, _SLOTS), jnp.int32),
                pltpu.SemaphoreType.DMA,
                pltpu.SemaphoreType.DMA,
                pltpu.SemaphoreType.DMA,
            ],
            **_PARAMS,
        )

    gctx = sweep(_B * _C)(ctxT, tail_c, ctx_idx)
    gtgt = sweep(_B)(tgtT, tail_t, tgt_idx)

    kb = pl.kernel(
        _kb_body,
        out_type=jax.ShapeDtypeStruct((_B,), jnp.float32),
        mesh=mesh,
        scratch_types=[
            pltpu.VMEM((_EB * _C, 128), jnp.float32),
            pltpu.VMEM((_EB, 128), jnp.float32),
            pltpu.VMEM((_BW,), jnp.float32),
            pltpu.VMEM((8,), jnp.float32),
            pltpu.SemaphoreType.DMA,
        ],
        **_PARAMS,
    )
    out = kb(gctx, gtgt, wb)
    return out.reshape(_B, 1)


# speedup vs baseline: 1.1450x; 1.1450x over previous
import jax
import jax.numpy as jnp
from jax import lax
from jax.experimental import pallas as pl
from jax.experimental.pallas import tpu as pltpu
from jax.experimental.pallas import tpu_sc as plsc

_B = 16384
_C = 5
_D = 64
_V = 1000000
_NW = 32
_BW = _B // _NW
_CHV = 512
_NCHUNK = _V // _CHV + 1
_TSLOT = (_NCHUNK + _NW - 1) // _NW
_BROW = 256
_BCAP = 240
_SLOTS = 64
_EB = 64

_PARAMS = dict(
    compiler_params=pltpu.CompilerParams(use_tc_tiling_on_sc=True,
                                         needs_layout_passes=False),
)


def _make_sweep(n_idx):
    nseg = n_idx // 4096
    dump = n_idx

    def body(tT, tailT, idx_hbm, gat_hbm,
             segbuf, bv, bp, counts, sv, sp, chunk, chunk_b, staging,
             posrow, sem, sem_a, sem_b):
        wid = lax.axis_index("s") * 2 + lax.axis_index("c")
        iota = lax.iota(jnp.int32, 16)
        czero = iota * 0
        lane0 = iota == 0

        for q in range(5):
            counts[pl.ds(16 * q, 16)] = czero

        def scan_seg(seg):
            pltpu.sync_copy(idx_hbm.at[pl.ds(seg * 4096, 4096)], segbuf)

            def scan_vec(i, carry):
                v16 = segbuf[pl.ds(16 * i, 16)]
                cid16 = lax.shift_right_logical(v16, 9)
                mine = lax.bitwise_and(cid16, 31) == wid
                pos16 = iota + (seg * 4096 + 16 * i)
                cnt = jnp.squeeze(lax.slice(
                    plsc.all_reduce_population_count(mine), (0,), (1,)))
                plsc.store_compressed(sv.at[pl.ds(0, 16)], v16, mask=mine)
                plsc.store_compressed(sp.at[pl.ds(0, 16)], pos16, mask=mine)

                def app(j, carry2):
                    vv = jnp.squeeze(lax.slice(
                        sv[pl.ds(j, 16)], (0,), (1,)))
                    pp = jnp.squeeze(lax.slice(
                        sp[pl.ds(j, 16)], (0,), (1,)))
                    t = jnp.minimum(
                        lax.shift_right_logical(vv, 14), _TSLOT - 1)
                    off = jnp.minimum(jnp.squeeze(lax.slice(
                        counts[pl.ds(t, 16)], (0,), (1,))), _BCAP)
                    okv = lane0 & (off < _BCAP)
                    slotpos = czero + (t * _BROW + off)
                    plsc.store_scatter(
                        bv, [slotpos],
                        czero + lax.bitwise_and(vv, _CHV - 1), mask=okv)
                    plsc.store_scatter(bp, [slotpos], czero + pp, mask=okv)
                    plsc.store_scatter(counts, [czero + t],
                                       czero + (off + 1), mask=okv)
                    return carry2

                lax.fori_loop(0, cnt, app, 0)
                return carry

            lax.fori_loop(0, 256, scan_vec, 0)

        for seg in range(nseg):
            scan_seg(seg)

        dump0 = dump + wid * _SLOTS

        def reset_posrow():
            for q in range(_SLOTS // 16):
                posrow[0, pl.ds(16 * q, 16)] = iota + (dump0 + 16 * q)

        reset_posrow()

        def process(t, buf, slot0, enable):
            tcnt = jnp.squeeze(lax.slice(
                counts[pl.ds(t, 16)], (0,), (1,)))
            tcnt = lax.select(enable, tcnt, 0)

            def lane(j, slot):
                vcol = czero + lax.bitwise_and(jnp.squeeze(lax.slice(
                    bv[pl.ds(t * _BROW + j, 16)], (0,), (1,))), _CHV - 1)
                pos = jnp.clip(jnp.squeeze(lax.slice(
                    bp[pl.ds(t * _BROW + j, 16)], (0,), (1,))),
                    0, n_idx - 1)
                for q in range(4):
                    cv = plsc.load_gather(buf, [iota + 16 * q, vcol])
                    staging[slot, pl.ds(16 * q, 16)] = cv
                plsc.store_scatter(posrow.at[0], [czero + slot],
                                   czero + pos, mask=lane0)
                slot = slot + 1

                @pl.when(slot == _SLOTS)
                def _():
                    pltpu.async_copy(staging, gat_hbm.at[posrow.at[0]],
                                     sem).wait()
                    reset_posrow()

                return lax.select(slot == _SLOTS, 0, slot)

            return lax.fori_loop(0, tcnt, lane, slot0)

        def start_main(t, buf, dsem):
            cid = wid + _NW * t

            @pl.when(cid < _NCHUNK - 1)
            def _():
                off = pl.multiple_of(cid * _CHV, 128)
                pltpu.async_copy(tT.at[:, pl.ds(off, _CHV)], buf, dsem)

        def wait_main(t, buf, dsem):
            cid = wid + _NW * t

            @pl.when(cid < _NCHUNK - 1)
            def _():
                pltpu.make_async_copy(
                    tT.at[:, pl.ds(0, _CHV)], buf, dsem).wait()

        start_main(0, chunk, sem_a)

        def do_pair(p, slot):
            t0 = 2 * p
            start_main(t0 + 1, chunk_b, sem_b)
            wait_main(t0, chunk, sem_a)
            slot = process(t0, chunk, slot,
                           wid + _NW * t0 < _NCHUNK - 1)
            start_main(t0 + 2, chunk, sem_a)
            wait_main(t0 + 1, chunk_b, sem_b)
            slot = process(t0 + 1, chunk_b, slot,
                           wid + _NW * (t0 + 1) < _NCHUNK - 1)
            return slot

        slot = lax.fori_loop(0, _TSLOT // 2, do_pair, 0)

        tail_cid = _NCHUNK - 1
        is_tail_owner = wid == tail_cid % _NW

        @pl.when(is_tail_owner)
        def _():
            pltpu.sync_copy(tailT, chunk.at[:, pl.ds(0, 128)])

        slot = process(_TSLOT - 1, chunk, slot, is_tail_owner)

        pltpu.async_copy(staging, gat_hbm.at[posrow.at[0]], sem).wait()

    return body


def _kb_body(gctx_hbm, gtgt_hbm, wb_hbm, out_hbm,
             ctx_rows, tgt_rows, out_v, wb_v, sem):
    wid = lax.axis_index("s") * 2 + lax.axis_index("c")
    base = wid * _BW
    iota = lax.iota(jnp.int32, 16)

    pltpu.sync_copy(wb_hbm, wb_v)
    tidx_pats = []
    wpats = []
    for v in range(20):
        k = iota + 16 * v
        tidx_pats.append(k // 5)
        wpats.append(plsc.load_gather(wb_v, [k % 5]))
    bvec = plsc.load_gather(wb_v, [iota * 0 + _C])

    for ch in range(_BW // _EB):
        cb = base + ch * _EB
        cp1 = pltpu.async_copy(gctx_hbm.at[pl.ds(cb * _C, _EB * _C)],
                               ctx_rows, sem)
        cp2 = pltpu.async_copy(gtgt_hbm.at[pl.ds(cb, _EB)], tgt_rows, sem)
        cp1.wait()
        cp2.wait()

        def body(e, carry):
            full_e = iota * 0 + e
            acc = jnp.zeros((16,), jnp.float32)
            for v in range(20):
                c_o, q = divmod(v, 4)
                cvec = ctx_rows[_C * e + c_o, pl.ds(16 * q, 16)]
                texp = plsc.load_gather(tgt_rows, [full_e, tidx_pats[v]])
                acc = acc + cvec * wpats[v] * texp
            s = jnp.zeros((16,), jnp.float32) + jnp.sum(acc)
            plsc.store_scatter(out_v, [full_e + ch * _EB], s,
                               mask=iota == 0)
            return carry

        lax.fori_loop(0, _EB, body, 0)

    for i in range(_BW // 16):
        x = out_v[pl.ds(16 * i, 16)] + bvec
        out_v[pl.ds(16 * i, 16)] = 1.0 / (1.0 + jnp.exp(-x))
    pltpu.sync_copy(out_v, out_hbm.at[pl.ds(base, _BW)])


@jax.jit
def kernel(context_input, target_input, context_table, target_table,
           W_dense, b_dense):
    ctx_idx = context_input.reshape(_B * _C)
    tgt_idx = target_input.reshape(_B)
    wb = jnp.concatenate([W_dense.reshape(_C), b_dense,
                          jnp.zeros((2,), jnp.float32)])
    ctxT = context_table.T
    tgtT = target_table.T
    tail_c = jnp.pad(ctxT[:, _V - _V % _CHV:], ((0, 0), (0, 64)))
    tail_t = jnp.pad(tgtT[:, _V - _V % _CHV:], ((0, 0), (0, 64)))

    mesh = plsc.VectorSubcoreMesh(core_axis_name="c", subcore_axis_name="s",
                                  num_cores=2, num_subcores=16)

    def sweep(n_idx):
        return pl.kernel(
            _make_sweep(n_idx),
            out_type=jax.ShapeDtypeStruct((n_idx + _NW * _SLOTS, 128),
                                          jnp.float32),
            mesh=mesh,
            scratch_types=[
                pltpu.VMEM((4096,), jnp.int32),
                pltpu.VMEM((_TSLOT * _BROW,), jnp.int32),
                pltpu.VMEM((_TSLOT * _BROW,), jnp.int32),
                pltpu.VMEM((80,), jnp.int32),
                pltpu.VMEM((32,), jnp.int32),
                pltpu.VMEM((32,), jnp.int32),
                pltpu.VMEM((_D, _CHV), jnp.float32),
                pltpu.VMEM((_D, _CHV), jnp.float32),
                pltpu.VMEM((_SLOTS, 128), jnp.float32),
                pltpu.VMEM((1, _SLOTS), jnp.int32),
                pltpu.SemaphoreType.DMA,
                pltpu.SemaphoreType.DMA,
                pltpu.SemaphoreType.DMA,
            ],
            **_PARAMS,
        )

    gctx = sweep(_B * _C)(ctxT, tail_c, ctx_idx)
    gtgt = sweep(_B)(tgtT, tail_t, tgt_idx)

    kb = pl.kernel(
        _kb_body,
        out_type=jax.ShapeDtypeStruct((_B,), jnp.float32),
        mesh=mesh,
        scratch_types=[
            pltpu.VMEM((_EB * _C, 128), jnp.float32),
            pltpu.VMEM((_EB, 128), jnp.float32),
            pltpu.VMEM((_BW,), jnp.float32),
            pltpu.VMEM((8,), jnp.float32),
            pltpu.SemaphoreType.DMA,
        ],
        **_PARAMS,
    )
    out = kb(gctx, gtgt, wb)
    return out.reshape(_B, 1)
